# baseline (speedup 1.0000x reference)
import functools

import jax
import jax.numpy as jnp
from jax import lax
from jax.experimental import pallas as pl
from jax.experimental.pallas import tpu as pltpu
from jax.experimental.pallas import tpu_sc as plsc

_LANES = 16
_CHUNKS = 4


def kernel(labels, table):
    B, = labels.shape
    V, D = table.shape
    info = plsc.get_sparse_core_info()
    NC, NS = info.num_cores, info.num_subcores
    NW = NC * NS
    b_per_w = B // NW
    n_groups = b_per_w // _LANES

    mesh = plsc.VectorSubcoreMesh(core_axis_name="c", subcore_axis_name="s")

    @functools.partial(
        pl.kernel,
        mesh=mesh,
        out_type=jax.ShapeDtypeStruct((B * D,), jnp.float32),
        compiler_params=pltpu.CompilerParams(
            needs_layout_passes=False,
            disable_bounds_checks=True,
            disable_semaphore_checks=True,
            skip_device_barrier=True,
        ),
        scratch_types=[
            pltpu.VMEM((V * D,), jnp.float32),
            pltpu.VMEM((b_per_w,), jnp.int32),
            pltpu.VMEM((b_per_w * D,), jnp.float32),
            pltpu.SemaphoreType.DMA,
        ],
    )
    def _emb(labels_hbm, table_hbm, out_hbm, table_v, idx_v, rows_v, sem):
        wid = lax.axis_index("s") * NC + lax.axis_index("c")
        base = wid * b_per_w
        pltpu.sync_copy(table_hbm, table_v)
        pltpu.sync_copy(labels_hbm.at[pl.ds(base, b_per_w)], idx_v)

        @plsc.parallel_loop(0, n_groups)
        def group_body(bg):
            gb = idx_v[pl.ds(bg * _LANES, _LANES)] * D
            for u in range(_LANES):
                gbase = gb[u]
                row = (bg * _LANES + u) * D
                for j in range(D // _LANES):
                    col = table_v[pl.ds(gbase + j * _LANES, _LANES)]
                    rows_v[pl.ds(row + j * _LANES, _LANES)] = col

        pltpu.async_copy(
            rows_v, out_hbm.at[pl.ds(base * D, b_per_w * D)], sem
        ).wait()

    labels_i32 = labels.astype(jnp.int32)
    table_flat = table.reshape(V * D)
    return _emb(labels_i32, table_flat).reshape(B, D)


# speedup vs baseline: 1.1438x; 1.1438x over previous
import functools

import jax
import jax.numpy as jnp
from jax import lax
from jax.experimental import pallas as pl
from jax.experimental.pallas import tpu as pltpu
from jax.experimental.pallas import tpu_sc as plsc

_IDX_CHUNK = 128


def kernel(labels, table):
    B, = labels.shape
    V, D = table.shape
    info = plsc.get_sparse_core_info()
    NC, NS = info.num_cores, info.num_subcores
    NW = NC * NS
    b_per_w = B // NW
    n_chunks = b_per_w // _IDX_CHUNK

    mesh = plsc.VectorSubcoreMesh(core_axis_name="c", subcore_axis_name="s")

    @functools.partial(
        pl.kernel,
        mesh=mesh,
        out_type=jax.ShapeDtypeStruct((B, D), jnp.float32),
        compiler_params=pltpu.CompilerParams(needs_layout_passes=False),
        scratch_types=[
            pltpu.VMEM_SHARED((V, D), jnp.float32),
            pltpu.VMEM((n_chunks, _IDX_CHUNK), jnp.int32),
            pltpu.VMEM((b_per_w, D), jnp.float32),
            pltpu.SemaphoreType.DMA,
        ],
    )
    def _emb(labels_hbm, table_hbm, out_hbm, table_s, idx_v, rows_v, sem):
        sid = lax.axis_index("s")
        wid = sid * NC + lax.axis_index("c")
        base = wid * b_per_w

        @pl.when(sid == 0)
        def _stage():
            pltpu.sync_copy(table_hbm, table_s)

        pltpu.sync_copy(labels_hbm.at[wid], idx_v)
        plsc.subcore_barrier()

        copies = []
        for c in range(n_chunks):
            copies.append(
                pltpu.async_copy(
                    table_s.at[idx_v.at[c]],
                    rows_v.at[pl.ds(c * _IDX_CHUNK, _IDX_CHUNK)],
                    sem,
                )
            )
        for cp in copies:
            cp.wait()
        pltpu.sync_copy(rows_v, out_hbm.at[pl.ds(base, b_per_w)])

    labels_r = labels.reshape(NW, n_chunks, _IDX_CHUNK).astype(jnp.int32)
    return _emb(labels_r, table)


# speedup vs baseline: 1.1963x; 1.0458x over previous
import functools

import jax
import jax.numpy as jnp
from jax import lax
from jax.experimental import pallas as pl
from jax.experimental.pallas import tpu as pltpu
from jax.experimental.pallas import tpu_sc as plsc

_IDX_CHUNK = 128


def kernel(labels, table):
    B, = labels.shape
    V, D = table.shape
    info = plsc.get_sparse_core_info()
    NC, NS = info.num_cores, info.num_subcores
    NW = NC * NS
    b_per_w = B // NW
    n_chunks = b_per_w // _IDX_CHUNK

    mesh = plsc.VectorSubcoreMesh(core_axis_name="c", subcore_axis_name="s")

    @functools.partial(
        pl.kernel,
        mesh=mesh,
        out_type=jax.ShapeDtypeStruct((B, D), jnp.float32),
        compiler_params=pltpu.CompilerParams(needs_layout_passes=False),
        scratch_types=[
            pltpu.VMEM_SHARED((V, D), jnp.float32),
            pltpu.VMEM((n_chunks, _IDX_CHUNK), jnp.int32),
            pltpu.VMEM((b_per_w, D), jnp.float32),
            pltpu.SemaphoreType.DMA,
            pltpu.SemaphoreType.DMA,
        ],
    )
    def _emb(labels_hbm, table_hbm, out_hbm, table_s, idx_v, rows_v, gsem, osem):
        sid = lax.axis_index("s")
        wid = sid * NC + lax.axis_index("c")
        base = wid * b_per_w

        @pl.when(sid == 0)
        def _stage():
            pltpu.sync_copy(table_hbm, table_s)

        pltpu.sync_copy(labels_hbm.at[wid], idx_v)
        plsc.subcore_barrier()

        gathers = [
            pltpu.async_copy(
                table_s.at[idx_v.at[c]],
                rows_v.at[pl.ds(c * _IDX_CHUNK, _IDX_CHUNK)],
                gsem,
            )
            for c in range(n_chunks)
        ]
        writes = []
        for c in range(n_chunks):
            gathers[c].wait()
            writes.append(
                pltpu.async_copy(
                    rows_v.at[pl.ds(c * _IDX_CHUNK, _IDX_CHUNK)],
                    out_hbm.at[pl.ds(base + c * _IDX_CHUNK, _IDX_CHUNK)],
                    osem,
                )
            )
        for w in writes:
            w.wait()

    labels_r = labels.reshape(NW, n_chunks, _IDX_CHUNK).astype(jnp.int32)
    return _emb(labels_r, table)
